# baseline (speedup 1.0000x reference)
import functools

import jax
import jax.numpy as jnp
from jax import lax
from jax.experimental import pallas as pl
from jax.experimental.pallas import tpu as pltpu
from jax.experimental.pallas import tpu_sc as plsc

N = 10000
F = 256
K = 16
PTS_PER_UNIT = 8
IDX_PER_UNIT = PTS_PER_UNIT * K
NUM_UNITS = N // PTS_PER_UNIT
LANES = 16
COLS = F // LANES
IDX_VREGS = IDX_PER_UNIT // LANES

_info = plsc.get_sparse_core_info()
NC, NS = _info.num_cores, _info.num_subcores
NW = NC * NS
UPW = -(-NUM_UNITS // NW)
UNITS_PAD = UPW * NW
BLK = 10
NBLK = UPW // BLK


def _reduce_unit(rows_v, out_v, slot):
    def point_body(p, carry):
        base = p * K
        accs = tuple(rows_v[base, pl.ds(c * LANES, LANES)] for c in range(COLS))

        def row_body(r, accs):
            return tuple(
                jnp.maximum(a, rows_v[base + r, pl.ds(c * LANES, LANES)])
                for c, a in enumerate(accs)
            )

        accs = lax.fori_loop(1, K, row_body, accs)
        for c in range(COLS):
            out_v[slot * PTS_PER_UNIT + p, pl.ds(c * LANES, LANES)] = accs[c]
        return carry

    lax.fori_loop(0, PTS_PER_UNIT, point_body, 0)


def _pool_kernel(feat_hbm, idx_hbm, out_hbm,
                 idx_all, idx0, idx1, rows0, rows1, out_blk, gsem0, gsem1):
    wid = lax.axis_index("s") * NC + lax.axis_index("c")
    ustart = wid * UPW

    def copy_idx_row(i, dst):
        for c in range(IDX_VREGS):
            dst[pl.ds(c * LANES, LANES)] = idx_all[i, pl.ds(c * LANES, LANES)]

    idx_refs = (idx0, idx1)
    rows_refs = (rows0, rows1)
    gsems = (gsem0, gsem1)

    pltpu.sync_copy(idx_hbm.at[pl.ds(ustart, UPW)], idx_all.at[pl.ds(0, UPW)])
    copy_idx_row(0, idx0)
    pltpu.async_copy(feat_hbm.at[idx0], rows0, gsem0)
    copy_idx_row(1, idx1)

    def blk_body(blk, carry):
        base_u = blk * BLK

        for k in range(BLK):
            i = base_u + k
            b = k % 2
            nb = (k + 1) % 2

            def issue_next():
                pltpu.async_copy(
                    feat_hbm.at[idx_refs[nb]], rows_refs[nb], gsems[nb])

            if k == BLK - 1:
                @pl.when(blk < NBLK - 1)
                def _():
                    issue_next()
            else:
                issue_next()

            pltpu.make_async_copy(
                feat_hbm.at[idx_refs[b]], rows_refs[b], gsems[b]).wait()
            copy_idx_row(i + 2, idx_refs[b])
            _reduce_unit(rows_refs[b], out_blk, k)

        @pl.when(ustart + base_u + BLK <= NUM_UNITS)
        def _():
            pltpu.sync_copy(
                out_blk,
                out_hbm.at[pl.ds((ustart + base_u) * PTS_PER_UNIT,
                                 BLK * PTS_PER_UNIT)])

        return carry

    lax.fori_loop(0, NBLK, blk_body, 0)


@jax.jit
def _pool(features, idx_pad):
    mesh = plsc.VectorSubcoreMesh(core_axis_name="c", subcore_axis_name="s")
    run = functools.partial(
        pl.kernel,
        mesh=mesh,
        out_type=jax.ShapeDtypeStruct((N, F), jnp.float32),
        scratch_types=[
            pltpu.VMEM((UPW + 2, IDX_PER_UNIT), jnp.int32),
            pltpu.VMEM((IDX_PER_UNIT,), jnp.int32),
            pltpu.VMEM((IDX_PER_UNIT,), jnp.int32),
            pltpu.VMEM((IDX_PER_UNIT, F), jnp.float32),
            pltpu.VMEM((IDX_PER_UNIT, F), jnp.float32),
            pltpu.VMEM((BLK * PTS_PER_UNIT, F), jnp.float32),
            pltpu.SemaphoreType.DMA,
            pltpu.SemaphoreType.DMA,
        ],
    )(_pool_kernel)
    return run(features, idx_pad)


def kernel(points, features, neighbor_indices):
    del points
    idx = neighbor_indices.astype(jnp.int32).reshape(NUM_UNITS, IDX_PER_UNIT)
    idx_pad = jnp.pad(idx, ((0, UNITS_PAD - NUM_UNITS), (0, 0)))
    return _pool(features, idx_pad)
